# Initial kernel scaffold; baseline (speedup 1.0000x reference)
import functools

import jax
import jax.numpy as jnp
from jax import lax
from jax.experimental import pallas as pl
from jax.experimental.pallas import tpu as pltpu
from jax.experimental.pallas import tpu_sc as plsc

N = 10000
E = 320000
NC = 2
NS = 16
NW = NC * NS
CH = 128
NCHUNK = 2560
EP = NCHUNK * CH
CPW = NCHUNK // NW
NPAD = 10016
ZR = NPAD // NS
OR_ = N // NS

_mesh = plsc.VectorSubcoreMesh(
    core_axis_name="c", subcore_axis_name="s", num_cores=NC, num_subcores=NS
)


def _sc_scatter(D):
    @functools.partial(
        pl.kernel,
        mesh=_mesh,
        out_type=jax.ShapeDtypeStruct((NC, N, D), jnp.float32),
        scratch_types=[
            pltpu.VMEM((CPW, CH), jnp.int32),
            pltpu.VMEM((CPW, CH), jnp.int32),
            pltpu.VMEM((2, CH, D), jnp.float32),
            pltpu.VMEM_SHARED((NPAD, D), jnp.float32),
            pltpu.SemaphoreType.DMA,
            pltpu.SemaphoreType.DMA,
        ],
    )
    def k(p_hbm, srcc_hbm, dstc_hbm, z_hbm, out_hbm, src_v, dst_v, rows_v, acc_sh, sem0, sem1):
        c = lax.axis_index("c")
        s = lax.axis_index("s")
        w = c * NS + s
        pltpu.sync_copy(z_hbm.at[pl.ds(s * ZR, ZR)], acc_sh.at[pl.ds(s * ZR, ZR)])
        pltpu.sync_copy(srcc_hbm.at[pl.ds(w * CPW, CPW)], src_v)
        pltpu.sync_copy(dstc_hbm.at[pl.ds(w * CPW, CPW)], dst_v)
        plsc.subcore_barrier()

        pltpu.async_copy(p_hbm.at[src_v.at[0]], rows_v.at[0], sem0)
        pltpu.async_copy(p_hbm.at[src_v.at[1]], rows_v.at[1], sem1)

        def step(i, carry):
            j0 = 2 * i
            for b in range(2):
                j = j0 + b
                sem = sem0 if b == 0 else sem1
                pltpu.make_async_copy(
                    p_hbm.at[src_v.at[j]], rows_v.at[b], sem
                ).wait()
                pltpu.sync_copy(rows_v.at[b], acc_sh.at[dst_v.at[j]], add=True)

                @pl.when(j + 2 < CPW)
                def _():
                    pltpu.async_copy(p_hbm.at[src_v.at[j + 2]], rows_v.at[b], sem)

            return carry

        lax.fori_loop(0, CPW // 2, step, 0)
        plsc.subcore_barrier()
        pltpu.sync_copy(
            acc_sh.at[pl.ds(s * OR_, OR_)], out_hbm.at[c, pl.ds(s * OR_, OR_)]
        )

    return k


@functools.partial(
    pl.kernel,
    mesh=_mesh,
    out_type=jax.ShapeDtypeStruct((NC, N, 8), jnp.float32),
    scratch_types=[
        pltpu.VMEM((CPW, CH), jnp.int32),
        pltpu.VMEM((CH, 8), jnp.float32),
        pltpu.VMEM_SHARED((NPAD, 8), jnp.float32),
    ],
)
def _sc_degree(dstc_hbm, ones_hbm, z_hbm, out_hbm, dst_v, ones_v, deg_sh):
    c = lax.axis_index("c")
    s = lax.axis_index("s")
    w = c * NS + s
    pltpu.sync_copy(z_hbm.at[pl.ds(s * ZR, ZR)], deg_sh.at[pl.ds(s * ZR, ZR)])
    pltpu.sync_copy(dstc_hbm.at[pl.ds(w * CPW, CPW)], dst_v)
    pltpu.sync_copy(ones_hbm, ones_v)
    plsc.subcore_barrier()

    def step(j, carry):
        pltpu.sync_copy(ones_v, deg_sh.at[dst_v.at[j]], add=True)
        return carry

    lax.fori_loop(0, CPW, step, 0)
    plsc.subcore_barrier()
    pltpu.sync_copy(
        deg_sh.at[pl.ds(s * OR_, OR_)], out_hbm.at[c, pl.ds(s * OR_, OR_)]
    )


def _tc_matmul(x, W):
    def f(x_ref, w_ref, o_ref):
        o_ref[...] = jnp.dot(x_ref[...], w_ref[...], preferred_element_type=jnp.float32)

    return pl.pallas_call(
        f, out_shape=jax.ShapeDtypeStruct((x.shape[0], W.shape[1]), jnp.float32)
    )(x, W)


def _tc_dinv_scale(h1, dega):
    def f(h_ref, d_ref, p_ref, dinv_ref):
        dinv8 = lax.rsqrt(d_ref[0] + d_ref[1] + 1.0)
        dinv_ref[...] = dinv8
        p_ref[...] = dinv8[:, 0:1] * h_ref[...]

    return pl.pallas_call(
        f,
        out_shape=(
            jax.ShapeDtypeStruct(h1.shape, jnp.float32),
            jax.ShapeDtypeStruct((N, 8), jnp.float32),
        ),
    )(h1, dega)


def _tc_mid(acc, P, dinv8, W, b):
    def f(a_ref, p_ref, d_ref, w_ref, b_ref, o_ref):
        d1 = d_ref[:, 0:1]
        h = d1 * (a_ref[0] + a_ref[1] + p_ref[...]) + b_ref[...]
        h = jnp.maximum(h, 0.0)
        o_ref[...] = d1 * jnp.dot(h, w_ref[...], preferred_element_type=jnp.float32)

    return pl.pallas_call(
        f, out_shape=jax.ShapeDtypeStruct((N, W.shape[1]), jnp.float32)
    )(acc, P, dinv8, W, b)


def _tc_final(acc, P, dinv8, b):
    def f(a_ref, p_ref, d_ref, b_ref, o_ref):
        h = d_ref[:, 0:1] * (a_ref[0] + a_ref[1] + p_ref[...]) + b_ref[...]
        o_ref[...] = jax.nn.sigmoid(h)

    return pl.pallas_call(
        f, out_shape=jax.ShapeDtypeStruct(P.shape, jnp.float32)
    )(acc, P, dinv8, b)


@jax.jit
def kernel(x, edge_index, W1, b1, W2, b2, W3, b3):
    src = edge_index[0].astype(jnp.int32)
    dst = edge_index[1].astype(jnp.int32)
    npad = EP - E
    ar = jnp.arange(npad, dtype=jnp.int32)
    srcc = jnp.concatenate([src, ar % N]).reshape(NCHUNK, CH)
    dstc = jnp.concatenate([dst, N + (ar % (NPAD - N))]).reshape(NCHUNK, CH)

    z8 = jnp.zeros((NPAD, 8), jnp.float32)
    ones8 = jnp.ones((CH, 8), jnp.float32)

    dega = _sc_degree(dstc, ones8, z8)
    h1 = _tc_matmul(x, W1)
    P1, dinv8 = _tc_dinv_scale(h1, dega)

    acc1 = _sc_scatter(64)(P1, srcc, dstc, jnp.zeros((NPAD, 64), jnp.float32))
    P2 = _tc_mid(acc1, P1, dinv8, W2, b1.reshape(1, -1))

    acc2 = _sc_scatter(32)(P2, srcc, dstc, jnp.zeros((NPAD, 32), jnp.float32))
    P3 = _tc_mid(acc2, P2, dinv8, W3, b2.reshape(1, -1))

    acc3 = _sc_scatter(16)(P3, srcc, dstc, jnp.zeros((NPAD, 16), jnp.float32))
    return _tc_final(acc3, P3, dinv8, b3.reshape(1, -1))


# speedup vs baseline: 10.0413x; 10.0413x over previous
import functools

import jax
import jax.numpy as jnp
from jax import lax
from jax.experimental import pallas as pl
from jax.experimental.pallas import tpu as pltpu
from jax.experimental.pallas import tpu_sc as plsc

N = 10000
E = 320000
LW = 128
HW = 32
NS = 16
CH = 128
NCHUNK = 2560
EP = NCHUNK * CH
CPW = NCHUNK // NS
EPW = CPW * CH
DPAD = 10112
NPH = 2560
ZR = NPH // NS
NH = 2500
OR_ = 152
L = 16

_mesh = plsc.VectorSubcoreMesh(
    core_axis_name="c", subcore_axis_name="s", num_cores=1, num_subcores=NS
)


def _unpack_indices(pk_v, src_v, dst_v):
    def row(j, carry):
        for k in range(CH // L):
            v = pk_v[pl.ds(j * CH + k * L, L)]
            d = lax.bitwise_and(v, 16383)
            par = lax.bitwise_and(v, 3)
            src_v[j, pl.ds(k * L, L)] = (
                lax.shift_left(lax.shift_right_logical(v, 14), 2) + par
            )
            dst_v[j, pl.ds(k * L, L)] = lax.shift_right_logical(d, 2)
        return carry

    lax.fori_loop(0, CPW, row, 0)


@functools.partial(
    pl.kernel,
    mesh=_mesh,
    out_type=jax.ShapeDtypeStruct((NH, LW), jnp.float32),
    scratch_types=[
        pltpu.VMEM((EPW,), jnp.int32),
        pltpu.VMEM((CPW, CH), jnp.int32),
        pltpu.VMEM((CPW, CH), jnp.int32),
        pltpu.VMEM((2, CH, LW), jnp.float32),
        pltpu.VMEM((8, LW), jnp.float32),
        pltpu.VMEM_SHARED((NPH, LW), jnp.float32),
        pltpu.SemaphoreType.DMA,
        pltpu.SemaphoreType.DMA,
    ],
)
def _sc_scatter(p_hbm, idxp_hbm, out_hbm, pk_v, src_v, dst_v, rows_v, zb_v, acc_sh, sem0, sem1):
    s = lax.axis_index("s")

    for k8 in range(8):
        for kk in range(LW // L):
            zb_v[k8, pl.ds(kk * L, L)] = jnp.zeros((L,), jnp.float32)

    def zrow(r, carry):
        pltpu.sync_copy(zb_v, acc_sh.at[pl.ds(s * ZR + r * 8, 8)])
        return carry

    lax.fori_loop(0, ZR // 8, zrow, 0)
    pltpu.sync_copy(idxp_hbm.at[pl.ds(s * EPW, EPW)], pk_v)
    _unpack_indices(pk_v, src_v, dst_v)
    plsc.subcore_barrier()

    pltpu.async_copy(p_hbm.at[src_v.at[0]], rows_v.at[0], sem0)
    pltpu.async_copy(p_hbm.at[src_v.at[1]], rows_v.at[1], sem1)

    def step(i, carry):
        j0 = 2 * i
        for b in range(2):
            j = j0 + b
            sem = sem0 if b == 0 else sem1
            pltpu.make_async_copy(p_hbm.at[src_v.at[j]], rows_v.at[b], sem).wait()
            pltpu.sync_copy(rows_v.at[b], acc_sh.at[dst_v.at[j]], add=True)

            @pl.when(j + 2 < CPW)
            def _():
                pltpu.async_copy(p_hbm.at[src_v.at[j + 2]], rows_v.at[b], sem)

        return carry

    lax.fori_loop(0, CPW // 2, step, 0)
    plsc.subcore_barrier()

    @pl.when(s < NS - 1)
    def _():
        pltpu.sync_copy(
            acc_sh.at[pl.ds(s * OR_, OR_)], out_hbm.at[pl.ds(s * OR_, OR_)]
        )

    @pl.when(s == NS - 1)
    def _():
        pltpu.sync_copy(
            acc_sh.at[pl.ds((NS - 1) * OR_, NH - (NS - 1) * OR_)],
            out_hbm.at[pl.ds((NS - 1) * OR_, NH - (NS - 1) * OR_)],
        )


def _tc_matmul(x, W):
    def f(x_ref, w_ref, o_ref):
        o_ref[...] = jnp.dot(x_ref[...], w_ref[...], preferred_element_type=jnp.float32)

    return pl.pallas_call(
        f, out_shape=jax.ShapeDtypeStruct((x.shape[0], W.shape[1]), jnp.float32)
    )(x, W)


def _pair_table(ph):
    if ph.shape[1] < HW:
        ph = jnp.concatenate(
            [ph, jnp.zeros((N, HW - ph.shape[1]), jnp.float32)], axis=1
        )
    z32 = jnp.zeros((N, HW), jnp.float32)
    parts = []
    for q in range(4):
        parts += [z32] * q + [ph] + [z32] * (3 - q)
    return jnp.concatenate(parts, axis=1)


def _tc_dinv_scale(h1, acc0):
    def f(h_ref, a_ref, ta_ref, tb_ref, dinv_ref):
        cnt = a_ref[:, 0:1]
        dinv8 = jnp.broadcast_to(lax.rsqrt(cnt + 1.0), (N, 8))
        dinv_ref[...] = dinv8
        ph = dinv8[:, 0:1] * h_ref[...]
        ta_ref[...] = _pair_table(ph[:, :HW])
        tb_ref[...] = _pair_table(ph[:, HW:])

    return pl.pallas_call(
        f,
        out_shape=(
            jax.ShapeDtypeStruct((N, 4 * LW), jnp.float32),
            jax.ShapeDtypeStruct((N, 4 * LW), jnp.float32),
            jax.ShapeDtypeStruct((N, 8), jnp.float32),
        ),
    )(h1, acc0)


def _tc_mid1(acc_a, acc_b, Pa, Pb, dinv8, W, b):
    def f(aa_ref, ab_ref, pa_ref, pb_ref, d_ref, w_ref, b_ref, o_ref):
        d1 = d_ref[:, 0:1]
        h = jnp.concatenate(
            [aa_ref[...] + pa_ref[:, :HW], ab_ref[...] + pb_ref[:, :HW]], axis=1
        )
        h = jnp.maximum(d1 * h + b_ref[...], 0.0)
        hw_ = d1 * jnp.dot(h, w_ref[...], preferred_element_type=jnp.float32)
        o_ref[...] = _pair_table(hw_)

    return pl.pallas_call(
        f, out_shape=jax.ShapeDtypeStruct((N, 4 * LW), jnp.float32)
    )(acc_a, acc_b, Pa, Pb, dinv8, W, b)


def _tc_mid2(acc, P, dinv8, W, b):
    def f(a_ref, p_ref, d_ref, w_ref, b_ref, o_ref):
        d1 = d_ref[:, 0:1]
        h = d1 * (a_ref[...] + p_ref[:, :HW]) + b_ref[...]
        h = jnp.maximum(h, 0.0)
        hw_ = d1 * jnp.dot(h, w_ref[...], preferred_element_type=jnp.float32)
        o_ref[...] = _pair_table(hw_)

    return pl.pallas_call(
        f, out_shape=jax.ShapeDtypeStruct((N, 4 * LW), jnp.float32)
    )(acc, P, dinv8, W, b)


def _tc_final(acc, P, dinv8, b):
    def f(a_ref, p_ref, d_ref, b_ref, o_ref):
        h = d_ref[:, 0:1] * (a_ref[:, :16] + p_ref[:, :16]) + b_ref[...]
        o_ref[...] = jax.nn.sigmoid(h)

    return pl.pallas_call(
        f, out_shape=jax.ShapeDtypeStruct((N, 16), jnp.float32)
    )(acc, P, dinv8, b)


@jax.jit
def kernel(x, edge_index, W1, b1, W2, b2, W3, b3):
    src = edge_index[0].astype(jnp.int32)
    dst = edge_index[1].astype(jnp.int32)
    npad = EP - E
    ar = jnp.arange(npad, dtype=jnp.int32)
    srcp = jnp.concatenate([src, ar % N])
    dstp = jnp.concatenate([dst, N + (ar % (DPAD - N))])
    idxp = (srcp << 14) | dstp

    def unpack(a):
        return a.reshape(N, HW)

    t0 = jnp.zeros((N, 4 * LW), jnp.float32)
    for q in range(4):
        t0 = t0.at[:, q * LW + q * HW].set(1.0)
    acc0 = _sc_scatter(t0.reshape(4 * N, LW), idxp)
    h1 = _tc_matmul(x, W1)
    T1a, T1b, dinv8 = _tc_dinv_scale(h1, unpack(acc0))

    acc1a = _sc_scatter(T1a.reshape(4 * N, LW), idxp)
    acc1b = _sc_scatter(T1b.reshape(4 * N, LW), idxp)
    T2 = _tc_mid1(unpack(acc1a), unpack(acc1b), T1a[:, :HW], T1b[:, :HW], dinv8, W2, b1.reshape(1, -1))

    acc2 = _sc_scatter(T2.reshape(4 * N, LW), idxp)
    T3 = _tc_mid2(unpack(acc2), T2[:, :HW], dinv8, W3, b2.reshape(1, -1))

    acc3 = _sc_scatter(T3.reshape(4 * N, LW), idxp)
    return _tc_final(unpack(acc3), T3[:, :16], dinv8, b3.reshape(1, -1))


# speedup vs baseline: 15.2246x; 1.5162x over previous
import functools

import jax
import jax.numpy as jnp
from jax import lax
from jax.experimental import pallas as pl
from jax.experimental.pallas import tpu as pltpu
from jax.experimental.pallas import tpu_sc as plsc

N = 10000
E = 320000
LW = 128
HW = 32
NC = 2
NS = 16
CH = 128
NCHUNK = 2560
EP = NCHUNK * CH
NW = NC * NS
CPW = NCHUNK // NW
EPW = CPW * CH
DPAD = 10112
NPH = 2560
ZR = NPH // NS
NH = 2500
OR_ = 152
L = 16

_mesh = plsc.VectorSubcoreMesh(
    core_axis_name="c", subcore_axis_name="s", num_cores=NC, num_subcores=NS
)


def _unpack_indices(pk_v, src_v, dst_v):
    def row(j, carry):
        for k in range(CH // L):
            v = pk_v[pl.ds(j * CH + k * L, L)]
            d = lax.bitwise_and(v, 16383)
            par = lax.bitwise_and(v, 3)
            src_v[j, pl.ds(k * L, L)] = (
                lax.shift_left(lax.shift_right_logical(v, 14), 2) + par
            )
            dst_v[j, pl.ds(k * L, L)] = lax.shift_right_logical(d, 2)
        return carry

    lax.fori_loop(0, CPW, row, 0)


@functools.partial(
    pl.kernel,
    mesh=_mesh,
    out_type=jax.ShapeDtypeStruct((NC, NH, LW), jnp.float32),
    scratch_types=[
        pltpu.VMEM((EPW,), jnp.int32),
        pltpu.VMEM((CPW, CH), jnp.int32),
        pltpu.VMEM((CPW, CH), jnp.int32),
        pltpu.VMEM((2, CH, LW), jnp.float32),
        pltpu.VMEM((8, LW), jnp.float32),
        pltpu.VMEM_SHARED((NPH, LW), jnp.float32),
        pltpu.SemaphoreType.DMA,
        pltpu.SemaphoreType.DMA,
    ],
)
def _sc_scatter(p_hbm, idxp_hbm, out_hbm, pk_v, src_v, dst_v, rows_v, zb_v, acc_sh, sem0, sem1):
    c = lax.axis_index("c")
    s = lax.axis_index("s")
    w = c * NS + s

    for k8 in range(8):
        for kk in range(LW // L):
            zb_v[k8, pl.ds(kk * L, L)] = jnp.zeros((L,), jnp.float32)

    def zrow(r, carry):
        pltpu.sync_copy(zb_v, acc_sh.at[pl.ds(s * ZR + r * 8, 8)])
        return carry

    lax.fori_loop(0, ZR // 8, zrow, 0)
    pltpu.sync_copy(idxp_hbm.at[pl.ds(w * EPW, EPW)], pk_v)
    _unpack_indices(pk_v, src_v, dst_v)
    plsc.subcore_barrier()

    pltpu.async_copy(p_hbm.at[src_v.at[0]], rows_v.at[0], sem0)
    pltpu.async_copy(p_hbm.at[src_v.at[1]], rows_v.at[1], sem1)

    def step(i, carry):
        j0 = 2 * i
        for b in range(2):
            j = j0 + b
            sem = sem0 if b == 0 else sem1
            pltpu.make_async_copy(p_hbm.at[src_v.at[j]], rows_v.at[b], sem).wait()
            pltpu.sync_copy(rows_v.at[b], acc_sh.at[dst_v.at[j]], add=True)

            @pl.when(j + 2 < CPW)
            def _():
                pltpu.async_copy(p_hbm.at[src_v.at[j + 2]], rows_v.at[b], sem)

        return carry

    lax.fori_loop(0, CPW // 2, step, 0)
    plsc.subcore_barrier()

    @pl.when(s < NS - 1)
    def _():
        pltpu.sync_copy(
            acc_sh.at[pl.ds(s * OR_, OR_)], out_hbm.at[c, pl.ds(s * OR_, OR_)]
        )

    @pl.when(s == NS - 1)
    def _():
        pltpu.sync_copy(
            acc_sh.at[pl.ds((NS - 1) * OR_, NH - (NS - 1) * OR_)],
            out_hbm.at[c, pl.ds((NS - 1) * OR_, NH - (NS - 1) * OR_)],
        )


def _tc_matmul(x, W):
    def f(x_ref, w_ref, o_ref):
        o_ref[...] = jnp.dot(x_ref[...], w_ref[...], preferred_element_type=jnp.float32)

    return pl.pallas_call(
        f, out_shape=jax.ShapeDtypeStruct((x.shape[0], W.shape[1]), jnp.float32)
    )(x, W)


def _pair_table(ph):
    if ph.shape[1] < HW:
        ph = jnp.concatenate(
            [ph, jnp.zeros((N, HW - ph.shape[1]), jnp.float32)], axis=1
        )
    z32 = jnp.zeros((N, HW), jnp.float32)
    parts = []
    for q in range(4):
        parts += [z32] * q + [ph] + [z32] * (3 - q)
    return jnp.concatenate(parts, axis=1)


def _tc_dinv_scale(h1, acc0, acc0b):
    def f(h_ref, a_ref, a2_ref, ta_ref, dinv_ref):
        cnt = a_ref[:, 0:1] + a2_ref[:, 0:1]
        dinv8 = jnp.broadcast_to(lax.rsqrt(cnt + 1.0), (N, 8))
        dinv_ref[...] = dinv8
        ta_ref[...] = _pair_table(dinv8[:, 0:1] * h_ref[:, :HW])

    return pl.pallas_call(
        f,
        out_shape=(
            jax.ShapeDtypeStruct((N, 4 * LW), jnp.float32),
            jax.ShapeDtypeStruct((N, 8), jnp.float32),
        ),
    )(h1, acc0, acc0b)


def _tc_scale_b(h1, dinv8):
    def f(h_ref, d_ref, tb_ref):
        tb_ref[...] = _pair_table(d_ref[:, 0:1] * h_ref[:, HW:])

    return pl.pallas_call(
        f, out_shape=jax.ShapeDtypeStruct((N, 4 * LW), jnp.float32)
    )(h1, dinv8)


def _tc_mid1(aa0, aa1, ab0, ab1, Pa, Pb, dinv8, W, b):
    def f(aa0_ref, aa1_ref, ab0_ref, ab1_ref, pa_ref, pb_ref, d_ref, w_ref, b_ref, o_ref):
        d1 = d_ref[:, 0:1]
        h = jnp.concatenate(
            [
                aa0_ref[...] + aa1_ref[...] + pa_ref[:, :HW],
                ab0_ref[...] + ab1_ref[...] + pb_ref[:, :HW],
            ],
            axis=1,
        )
        h = jnp.maximum(d1 * h + b_ref[...], 0.0)
        hw_ = d1 * jnp.dot(h, w_ref[...], preferred_element_type=jnp.float32)
        o_ref[...] = _pair_table(hw_)

    return pl.pallas_call(
        f,
        out_shape=jax.ShapeDtypeStruct((N, 4 * LW), jnp.float32),
        compiler_params=pltpu.CompilerParams(vmem_limit_bytes=100 * 1024 * 1024),
    )(aa0, aa1, ab0, ab1, Pa, Pb, dinv8, W, b)


def _tc_mid2(a0, a1, P, dinv8, W, b):
    def f(a_ref, a2_ref, p_ref, d_ref, w_ref, b_ref, o_ref):
        d1 = d_ref[:, 0:1]
        h = d1 * (a_ref[...] + a2_ref[...] + p_ref[:, :HW]) + b_ref[...]
        h = jnp.maximum(h, 0.0)
        hw_ = d1 * jnp.dot(h, w_ref[...], preferred_element_type=jnp.float32)
        o_ref[...] = _pair_table(hw_)

    return pl.pallas_call(
        f, out_shape=jax.ShapeDtypeStruct((N, 4 * LW), jnp.float32)
    )(a0, a1, P, dinv8, W, b)


def _tc_final(a0, a1, P, dinv8, b):
    def f(a_ref, a2_ref, p_ref, d_ref, b_ref, o_ref):
        h = d_ref[:, 0:1] * (a_ref[:, :16] + a2_ref[:, :16] + p_ref[:, :16]) + b_ref[...]
        o_ref[...] = jax.nn.sigmoid(h)

    return pl.pallas_call(
        f, out_shape=jax.ShapeDtypeStruct((N, 16), jnp.float32)
    )(a0, a1, P, dinv8, b)


@jax.jit
def kernel(x, edge_index, W1, b1, W2, b2, W3, b3):
    src = edge_index[0].astype(jnp.int32)
    dst = edge_index[1].astype(jnp.int32)
    npad = EP - E
    ar = jnp.arange(npad, dtype=jnp.int32)
    srcp = jnp.concatenate([src, ar % N])
    dstp = jnp.concatenate([dst, N + (ar % (DPAD - N))])
    idxp = (srcp << 14) | dstp

    def unpack(a):
        return a[0].reshape(N, HW), a[1].reshape(N, HW)

    t0 = jnp.zeros((N, 4 * LW), jnp.float32)
    for q in range(4):
        t0 = t0.at[:, q * LW + q * HW].set(1.0)
    acc0 = _sc_scatter(t0.reshape(4 * N, LW), idxp)
    h1 = _tc_matmul(x, W1)
    T1a, dinv8 = _tc_dinv_scale(h1, *unpack(acc0))
    T1b = _tc_scale_b(h1, dinv8)

    acc1a = _sc_scatter(T1a.reshape(4 * N, LW), idxp)
    acc1b = _sc_scatter(T1b.reshape(4 * N, LW), idxp)
    T2 = _tc_mid1(*unpack(acc1a), *unpack(acc1b), T1a[:, :HW], T1b[:, :HW], dinv8, W2, b1.reshape(1, -1))

    acc2 = _sc_scatter(T2.reshape(4 * N, LW), idxp)
    T3 = _tc_mid2(*unpack(acc2), T2[:, :HW], dinv8, W3, b2.reshape(1, -1))

    acc3 = _sc_scatter(T3.reshape(4 * N, LW), idxp)
    return _tc_final(*unpack(acc3), T3[:, :16], dinv8, b3.reshape(1, -1))
